# baseline (speedup 1.0000x reference)
import jax
import jax.numpy as jnp
from jax import lax
from jax.experimental import pallas as pl
from jax.experimental.pallas import tpu as pltpu
from jax.experimental.pallas import tpu_sc as plsc

NC = 2
NS = 16
L = 16
CHUNK = 128
NBUF = 2
BLK = 128
ALPHA = 0.1


def _mlp_body(x_ref, w0t_ref, b0_ref, w1t_ref, b1_ref, os_ref):
    h1 = jnp.dot(x_ref[...], w0t_ref[...], preferred_element_type=jnp.float32)
    h1 = jnp.maximum(h1 + b0_ref[...], 0.0)
    h2 = jnp.dot(h1, w1t_ref[...], preferred_element_type=jnp.float32)
    out = jnp.maximum(h2 + b1_ref[...], 0.0)
    dh = out.shape[-1] // 2
    os_ref[0] = out[:, :dh]
    os_ref[1] = out[:, dh:]


def _mlp(x, w0t, b0, w1t, b1, blk, n_acc):
    n, d = x.shape
    h = w0t.shape[1]
    grid = n // blk
    return pl.pallas_call(
        _mlp_body,
        grid=(grid,),
        in_specs=[
            pl.BlockSpec((blk, d), lambda i: (i, 0)),
            pl.BlockSpec((d, h), lambda i: (0, 0)),
            pl.BlockSpec((1, h), lambda i: (0, 0)),
            pl.BlockSpec((h, d), lambda i: (0, 0)),
            pl.BlockSpec((1, d), lambda i: (0, 0)),
        ],
        out_specs=pl.BlockSpec((NC, blk, d // 2), lambda i: (0, i, 0)),
        out_shape=jax.ShapeDtypeStruct((NC, n_acc, d // 2), jnp.float32),
    )(x, w0t, b0, w1t, b1)


def _edge_loop(table_sh, acc_sh, src_view, src_ring, dst_v, bufs, sems, isem, k,
               deg=None):
    pltpu.async_copy(src_view.at[pl.ds(0, NBUF)], src_ring, isem)
    plsc.subcore_barrier()

    def step(i, carry):
        j = i * NBUF
        pltpu.make_async_copy(src_view.at[pl.ds(0, NBUF)], src_ring, isem).wait()
        descs = [
            pltpu.async_copy(table_sh.at[src_ring.at[b]], bufs[b], sems[b])
            for b in range(NBUF)
        ]
        if deg is not None:
            ones_v, deg_sh, c = deg
            pltpu.sync_copy(ones_v, deg_sh.at[dst_v.at[j + c]], add=True)
        for b in range(NBUF):
            descs[b].wait()
            pltpu.sync_copy(bufs[b], acc_sh.at[dst_v.at[j + b]], add=True)
        jn = lax.min(j + NBUF, k - NBUF)
        pltpu.async_copy(src_view.at[pl.ds(jn, NBUF)], src_ring, isem)
        return carry

    lax.fori_loop(0, k // NBUF, step, 0)
    pltpu.make_async_copy(src_view.at[pl.ds(0, NBUF)], src_ring, isem).wait()
    plsc.subcore_barrier()


def _blend(abuf, hbuf, d0buf, d1buf):
    def row(r, carry):
        dv = d0buf[r, pl.ds(0, L)] + d1buf[r, pl.ds(0, L)] + 1.0
        rinv = (1.0 - ALPHA) / dv
        for q in range(4):
            sl = pl.ds(q * L, L)
            abuf[r, sl] = abuf[r, sl] * rinv + ALPHA * hbuf[r, sl]
        return carry

    lax.fori_loop(0, BLK, row, 0)


def _make_agg1(n, dh, n_acc, k):
    rpt = n_acc // NS
    mesh = plsc.VectorSubcoreMesh(core_axis_name="c", subcore_axis_name="s")
    out_type = [
        jax.ShapeDtypeStruct((NC, n_acc, dh), jnp.float32),
        jax.ShapeDtypeStruct((NC, n_acc, L), jnp.float32),
    ]
    scratch = [
        pltpu.VMEM((NBUF, CHUNK), jnp.int32),
        pltpu.VMEM((k, CHUNK), jnp.int32),
        pltpu.VMEM((CHUNK, dh), jnp.float32),
        pltpu.VMEM((CHUNK, dh), jnp.float32),
        pltpu.VMEM((CHUNK, L), jnp.float32),
        pltpu.VMEM_SHARED((n_acc, dh), jnp.float32),
        pltpu.VMEM_SHARED((n, dh), jnp.float32),
        pltpu.VMEM_SHARED((n_acc, L), jnp.float32),
        pltpu.SemaphoreType.DMA,
        pltpu.SemaphoreType.DMA,
        pltpu.SemaphoreType.DMA,
    ]

    def body(h_hbm, src_hbm, dst_hbm, zerol_hbm, ones_hbm, p_hbm, deg_hbm,
             src_ring, dst_v, buf_a, buf_b, ones_v, acc_sh, table_sh, deg_sh,
             sem_a, sem_b, isem):
        c = lax.axis_index("c")
        s = lax.axis_index("s")
        @pl.when(s == 0)
        def _():
            pltpu.sync_copy(h_hbm.at[c, pl.ds(0, n)], table_sh)
        pltpu.sync_copy(dst_hbm.at[s], dst_v)
        row0 = s * rpt
        rows = pl.ds(row0, rpt)
        pltpu.sync_copy(h_hbm.at[c, rows], acc_sh.at[rows])
        pltpu.sync_copy(zerol_hbm.at[rows], deg_sh.at[rows])
        pltpu.sync_copy(ones_hbm, ones_v)

        _edge_loop(table_sh, acc_sh, src_hbm.at[s], src_ring, dst_v,
                   [buf_a, buf_b], [sem_a, sem_b], isem, k,
                   deg=(ones_v, deg_sh, c))

        pltpu.sync_copy(acc_sh.at[rows], p_hbm.at[c, rows])
        pltpu.sync_copy(deg_sh.at[rows], deg_hbm.at[c, rows])

    return pl.kernel(
        body,
        out_type=out_type,
        mesh=mesh,
        scratch_types=scratch,
        compiler_params=pltpu.CompilerParams(use_tc_tiling_on_sc=False),
    )


def _make_l2(n, dh, n_acc, k):
    rpt = n_acc // NS
    nblk = rpt // BLK
    mesh = plsc.VectorSubcoreMesh(core_axis_name="c", subcore_axis_name="s")
    out_type = [jax.ShapeDtypeStruct((NC, n_acc, dh), jnp.float32)]
    scratch = [
        pltpu.VMEM((NBUF, CHUNK), jnp.int32),
        pltpu.VMEM((k, CHUNK), jnp.int32),
        pltpu.VMEM((CHUNK, dh), jnp.float32),
        pltpu.VMEM((CHUNK, dh), jnp.float32),
        pltpu.VMEM((BLK, L), jnp.float32),
        pltpu.VMEM((BLK, L), jnp.float32),
        pltpu.VMEM_SHARED((n_acc, dh), jnp.float32),
        pltpu.VMEM_SHARED((n_acc, dh), jnp.float32),
        pltpu.SemaphoreType.DMA,
        pltpu.SemaphoreType.DMA,
        pltpu.SemaphoreType.DMA,
    ]

    def body(p_hbm, deg_hbm, h_hbm, src_hbm, dst_hbm, out_hbm,
             src_ring, dst_v, buf_a, buf_b, d0buf, d1buf, acc_sh, table_sh,
             sem_a, sem_b, isem):
        c = lax.axis_index("c")
        s = lax.axis_index("s")
        pltpu.sync_copy(dst_hbm.at[s], dst_v)
        row0 = s * rpt
        for t in range(nblk):
            rows = pl.ds(row0 + t * BLK, BLK)
            pltpu.sync_copy(p_hbm.at[c, rows], buf_a)
            pltpu.sync_copy(h_hbm.at[c, rows], buf_b)
            pltpu.sync_copy(deg_hbm.at[0, rows], d0buf)
            pltpu.sync_copy(deg_hbm.at[1, rows], d1buf)
            _blend(buf_a, buf_b, d0buf, d1buf)
            pltpu.sync_copy(buf_a, table_sh.at[rows])
            pltpu.sync_copy(buf_a, acc_sh.at[rows])

        _edge_loop(table_sh, acc_sh, src_hbm.at[s], src_ring, dst_v,
                   [buf_a, buf_b], [sem_a, sem_b], isem, k)

        for t in range(nblk):
            rows = pl.ds(row0 + t * BLK, BLK)
            pltpu.sync_copy(acc_sh.at[rows], buf_a)
            pltpu.sync_copy(h_hbm.at[c, rows], buf_b)
            pltpu.sync_copy(deg_hbm.at[0, rows], d0buf)
            pltpu.sync_copy(deg_hbm.at[1, rows], d1buf)
            _blend(buf_a, buf_b, d0buf, d1buf)
            pltpu.sync_copy(buf_a, out_hbm.at[c, rows])

    return pl.kernel(
        body,
        out_type=out_type,
        mesh=mesh,
        scratch_types=scratch,
        compiler_params=pltpu.CompilerParams(use_tc_tiling_on_sc=False),
    )


def kernel(x, edge_index, W0, b0, W1, b1):
    n, d = x.shape
    e = edge_index.shape[1]
    dh = d // 2

    n_acc = -(-(n + 1) // (NS * BLK)) * (NS * BLK)

    blk = 1000 if n % 1000 == 0 else 8
    h_split = _mlp(x, W0.T, b0.reshape(1, -1), W1.T, b1.reshape(1, -1), blk, n_acc)

    per_xfer = NS * CHUNK
    k = -(-e // per_xfer)
    k += k % NBUF
    e_pad = k * per_xfer
    pad = e_pad - e
    src = jnp.concatenate([edge_index[0], jnp.zeros((pad,), jnp.int32)])
    dst = jnp.concatenate([edge_index[1], jnp.full((pad,), n, jnp.int32)])
    src_p = src.reshape(NS, k, CHUNK)
    dst_p = dst.reshape(NS, k, CHUNK)

    zeros_l = jnp.zeros((n_acc, L), jnp.float32)
    ones_l = jnp.ones((CHUNK, L), jnp.float32)

    p1, degp = _make_agg1(n, dh, n_acc, k)(h_split, src_p, dst_p, zeros_l, ones_l)
    (x2s,) = _make_l2(n, dh, n_acc, k)(p1, degp, h_split, src_p, dst_p)
    return jnp.concatenate([x2s[0, :n], x2s[1, :n]], axis=-1)


# speedup vs baseline: 1.0399x; 1.0399x over previous
import jax
import jax.numpy as jnp
from jax import lax
from jax.experimental import pallas as pl
from jax.experimental.pallas import tpu as pltpu
from jax.experimental.pallas import tpu_sc as plsc

NC = 2
NS = 16
L = 16
CHUNK = 128
NBUF = 2
BLK = 128
ALPHA = 0.1


def _mlp_body(x_ref, w0t_ref, b0_ref, w1t_ref, b1_ref, os_ref):
    h1 = jnp.dot(x_ref[...], w0t_ref[...], preferred_element_type=jnp.float32)
    h1 = jnp.maximum(h1 + b0_ref[...], 0.0)
    h2 = jnp.dot(h1, w1t_ref[...], preferred_element_type=jnp.float32)
    out = jnp.maximum(h2 + b1_ref[...], 0.0)
    dh = out.shape[-1] // 2
    os_ref[0] = out[:, :dh]
    os_ref[1] = out[:, dh:]


def _mlp(x, w0t, b0, w1t, b1, blk, n_acc):
    n, d = x.shape
    h = w0t.shape[1]
    grid = n // blk
    return pl.pallas_call(
        _mlp_body,
        grid=(grid,),
        in_specs=[
            pl.BlockSpec((blk, d), lambda i: (i, 0)),
            pl.BlockSpec((d, h), lambda i: (0, 0)),
            pl.BlockSpec((1, h), lambda i: (0, 0)),
            pl.BlockSpec((h, d), lambda i: (0, 0)),
            pl.BlockSpec((1, d), lambda i: (0, 0)),
        ],
        out_specs=pl.BlockSpec((NC, blk, d // 2), lambda i: (0, i, 0)),
        out_shape=jax.ShapeDtypeStruct((NC, n_acc, d // 2), jnp.float32),
    )(x, w0t, b0, w1t, b1)


def _edge_loop(table_sh, acc_sh, src_view, src_ring, dst_v, bufs, sems, isem, k,
               deg=None):
    pltpu.async_copy(src_view.at[pl.ds(0, NBUF)], src_ring, isem)
    plsc.subcore_barrier()

    def step(i, carry):
        j = i * NBUF
        pltpu.make_async_copy(src_view.at[pl.ds(0, NBUF)], src_ring, isem).wait()
        descs = [
            pltpu.async_copy(table_sh.at[src_ring.at[b]], bufs[b], sems[b])
            for b in range(NBUF)
        ]
        if deg is not None:
            ones_v, deg_sh, c = deg
            pltpu.sync_copy(ones_v, deg_sh.at[dst_v.at[j + c]], add=True)
        for b in range(NBUF):
            descs[b].wait()
            pltpu.sync_copy(bufs[b], acc_sh.at[dst_v.at[j + b]], add=True)
        jn = lax.min(j + NBUF, k - NBUF)
        pltpu.async_copy(src_view.at[pl.ds(jn, NBUF)], src_ring, isem)
        return carry

    lax.fori_loop(0, k // NBUF, step, 0)
    pltpu.make_async_copy(src_view.at[pl.ds(0, NBUF)], src_ring, isem).wait()
    plsc.subcore_barrier()


def _blend(abuf, hbuf, wbuf, dh):
    unroll = 4
    nq = dh // L

    def row(r, carry):
        for rr in range(unroll):
            ri = r * unroll + rr
            w = wbuf[ri, pl.ds(0, L)]
            for q in range(nq):
                sl = pl.ds(q * L, L)
                abuf[ri, sl] = abuf[ri, sl] * w + ALPHA * hbuf[ri, sl]
        return carry

    lax.fori_loop(0, BLK // unroll, row, 0)


def _winv_body(d_ref, o_ref):
    o_ref[...] = (1.0 - ALPHA) / (d_ref[0] + d_ref[1] + 1.0)


def _winv(degp):
    _, n_acc, l = degp.shape
    return pl.pallas_call(
        _winv_body,
        out_shape=jax.ShapeDtypeStruct((n_acc, l), jnp.float32),
    )(degp)


def _make_agg1(n, dh, n_acc, k):
    rpt = n_acc // NS
    mesh = plsc.VectorSubcoreMesh(core_axis_name="c", subcore_axis_name="s")
    out_type = [
        jax.ShapeDtypeStruct((NC, n_acc, dh), jnp.float32),
        jax.ShapeDtypeStruct((NC, n_acc, L), jnp.float32),
    ]
    scratch = [
        pltpu.VMEM((NBUF, CHUNK), jnp.int32),
        pltpu.VMEM((k, CHUNK), jnp.int32),
        pltpu.VMEM((CHUNK, dh), jnp.float32),
        pltpu.VMEM((CHUNK, dh), jnp.float32),
        pltpu.VMEM((CHUNK, L), jnp.float32),
        pltpu.VMEM_SHARED((n_acc, dh), jnp.float32),
        pltpu.VMEM_SHARED((n, dh), jnp.float32),
        pltpu.VMEM_SHARED((n_acc, L), jnp.float32),
        pltpu.SemaphoreType.DMA,
        pltpu.SemaphoreType.DMA,
        pltpu.SemaphoreType.DMA,
    ]

    def body(h_hbm, src_hbm, dst_hbm, zerol_hbm, ones_hbm, p_hbm, deg_hbm,
             src_ring, dst_v, buf_a, buf_b, ones_v, acc_sh, table_sh, deg_sh,
             sem_a, sem_b, isem):
        c = lax.axis_index("c")
        s = lax.axis_index("s")
        @pl.when(s == 0)
        def _():
            pltpu.sync_copy(h_hbm.at[c, pl.ds(0, n)], table_sh)
        pltpu.sync_copy(dst_hbm.at[s], dst_v)
        row0 = s * rpt
        rows = pl.ds(row0, rpt)
        pltpu.sync_copy(h_hbm.at[c, rows], acc_sh.at[rows])
        pltpu.sync_copy(zerol_hbm.at[rows], deg_sh.at[rows])
        pltpu.sync_copy(ones_hbm, ones_v)

        _edge_loop(table_sh, acc_sh, src_hbm.at[s], src_ring, dst_v,
                   [buf_a, buf_b], [sem_a, sem_b], isem, k,
                   deg=(ones_v, deg_sh, c))

        pltpu.sync_copy(acc_sh.at[rows], p_hbm.at[c, rows])
        pltpu.sync_copy(deg_sh.at[rows], deg_hbm.at[c, rows])

    return pl.kernel(
        body,
        out_type=out_type,
        mesh=mesh,
        scratch_types=scratch,
        compiler_params=pltpu.CompilerParams(use_tc_tiling_on_sc=False),
    )


def _make_l2(n, dh, n_acc, k):
    rpt = n_acc // NS
    nblk = rpt // BLK
    mesh = plsc.VectorSubcoreMesh(core_axis_name="c", subcore_axis_name="s")
    out_type = [jax.ShapeDtypeStruct((NC, n_acc, dh), jnp.float32)]
    scratch = [
        pltpu.VMEM((NBUF, CHUNK), jnp.int32),
        pltpu.VMEM((k, CHUNK), jnp.int32),
        pltpu.VMEM((CHUNK, dh), jnp.float32),
        pltpu.VMEM((CHUNK, dh), jnp.float32),
        pltpu.VMEM((BLK, L), jnp.float32),
        pltpu.VMEM_SHARED((n_acc, dh), jnp.float32),
        pltpu.VMEM_SHARED((n_acc, dh), jnp.float32),
        pltpu.SemaphoreType.DMA,
        pltpu.SemaphoreType.DMA,
        pltpu.SemaphoreType.DMA,
    ]

    def body(p_hbm, w_hbm, h_hbm, src_hbm, dst_hbm, out_hbm,
             src_ring, dst_v, buf_a, buf_b, wbuf, acc_sh, table_sh,
             sem_a, sem_b, isem):
        c = lax.axis_index("c")
        s = lax.axis_index("s")
        pltpu.sync_copy(dst_hbm.at[s], dst_v)
        row0 = s * rpt
        for t in range(nblk):
            rows = pl.ds(row0 + t * BLK, BLK)
            pltpu.sync_copy(p_hbm.at[c, rows], buf_a)
            pltpu.sync_copy(h_hbm.at[c, rows], buf_b)
            pltpu.sync_copy(w_hbm.at[rows], wbuf)
            _blend(buf_a, buf_b, wbuf, dh)
            pltpu.sync_copy(buf_a, table_sh.at[rows])
            pltpu.sync_copy(buf_a, acc_sh.at[rows])

        _edge_loop(table_sh, acc_sh, src_hbm.at[s], src_ring, dst_v,
                   [buf_a, buf_b], [sem_a, sem_b], isem, k)

        for t in range(nblk):
            rows = pl.ds(row0 + t * BLK, BLK)
            pltpu.sync_copy(acc_sh.at[rows], buf_a)
            pltpu.sync_copy(h_hbm.at[c, rows], buf_b)
            pltpu.sync_copy(w_hbm.at[rows], wbuf)
            _blend(buf_a, buf_b, wbuf, dh)
            pltpu.sync_copy(buf_a, out_hbm.at[c, rows])

    return pl.kernel(
        body,
        out_type=out_type,
        mesh=mesh,
        scratch_types=scratch,
        compiler_params=pltpu.CompilerParams(use_tc_tiling_on_sc=False),
    )


def kernel(x, edge_index, W0, b0, W1, b1):
    n, d = x.shape
    e = edge_index.shape[1]
    dh = d // 2

    n_acc = -(-(n + 1) // (NS * BLK)) * (NS * BLK)

    blk = 1000 if n % 1000 == 0 else 8
    h_split = _mlp(x, W0.T, b0.reshape(1, -1), W1.T, b1.reshape(1, -1), blk, n_acc)

    per_xfer = NS * CHUNK
    k = -(-e // per_xfer)
    k += k % NBUF
    e_pad = k * per_xfer
    pad = e_pad - e
    src = jnp.concatenate([edge_index[0], jnp.zeros((pad,), jnp.int32)])
    dst = jnp.concatenate([edge_index[1], jnp.full((pad,), n, jnp.int32)])
    src_p = src.reshape(NS, k, CHUNK)
    dst_p = dst.reshape(NS, k, CHUNK)

    zeros_l = jnp.zeros((n_acc, L), jnp.float32)
    ones_l = jnp.ones((CHUNK, L), jnp.float32)

    p1, degp = _make_agg1(n, dh, n_acc, k)(h_split, src_p, dst_p, zeros_l, ones_l)
    winv = _winv(degp)
    (x2s,) = _make_l2(n, dh, n_acc, k)(p1, winv, h_split, src_p, dst_p)
    return jnp.concatenate([x2s[0, :n], x2s[1, :n]], axis=-1)


# speedup vs baseline: 1.0615x; 1.0208x over previous
import jax
import jax.numpy as jnp
from jax import lax
from jax.experimental import pallas as pl
from jax.experimental.pallas import tpu as pltpu
from jax.experimental.pallas import tpu_sc as plsc

NC = 2
NS = 16
L = 16
CHUNK = 128
NBUF = 2
BLK = 128
ALPHA = 0.1


def _mlp_body(x_ref, w0t_ref, b0_ref, w1t_ref, b1_ref, os_ref):
    h1 = jnp.dot(x_ref[...], w0t_ref[...], preferred_element_type=jnp.float32)
    h1 = jnp.maximum(h1 + b0_ref[...], 0.0)
    h2 = jnp.dot(h1, w1t_ref[...], preferred_element_type=jnp.float32)
    out = jnp.maximum(h2 + b1_ref[...], 0.0)
    dh = out.shape[-1] // 2
    os_ref[0] = out[:, :dh]
    os_ref[1] = out[:, dh:]


def _mlp(x, w0t, b0, w1t, b1, blk, n_acc):
    n, d = x.shape
    h = w0t.shape[1]
    grid = n // blk
    return pl.pallas_call(
        _mlp_body,
        grid=(grid,),
        in_specs=[
            pl.BlockSpec((blk, d), lambda i: (i, 0)),
            pl.BlockSpec((d, h), lambda i: (0, 0)),
            pl.BlockSpec((1, h), lambda i: (0, 0)),
            pl.BlockSpec((h, d), lambda i: (0, 0)),
            pl.BlockSpec((1, d), lambda i: (0, 0)),
        ],
        out_specs=pl.BlockSpec((NC, blk, d // 2), lambda i: (0, i, 0)),
        out_shape=jax.ShapeDtypeStruct((NC, n_acc, d // 2), jnp.float32),
    )(x, w0t, b0, w1t, b1)


def _edge_loop(table_sh, acc_sh, src_view, src_ring, dst_v, bufs, sems, isem, k,
               deg=None):
    pltpu.async_copy(src_view.at[pl.ds(0, NBUF)], src_ring, isem)
    plsc.subcore_barrier()

    def step(i, carry):
        j = i * NBUF
        pltpu.make_async_copy(src_view.at[pl.ds(0, NBUF)], src_ring, isem).wait()
        descs = [
            pltpu.async_copy(table_sh.at[src_ring.at[b]], bufs[b], sems[b])
            for b in range(NBUF)
        ]
        if deg is not None:
            ones_v, deg_sh, c = deg
            pltpu.sync_copy(ones_v, deg_sh.at[dst_v.at[j + c]], add=True)
        for b in range(NBUF):
            descs[b].wait()
            pltpu.sync_copy(bufs[b], acc_sh.at[dst_v.at[j + b]], add=True)
        jn = lax.min(j + NBUF, k - NBUF)
        pltpu.async_copy(src_view.at[pl.ds(jn, NBUF)], src_ring, isem)
        return carry

    lax.fori_loop(0, k // NBUF, step, 0)
    pltpu.make_async_copy(src_view.at[pl.ds(0, NBUF)], src_ring, isem).wait()
    plsc.subcore_barrier()


def _blend(abuf, hbuf, wbuf, dh):
    unroll = 4
    nq = dh // L

    def row(r, carry):
        for rr in range(unroll):
            ri = r * unroll + rr
            w = wbuf[ri, pl.ds(0, L)]
            for q in range(nq):
                sl = pl.ds(q * L, L)
                abuf[ri, sl] = abuf[ri, sl] * w + ALPHA * hbuf[ri, sl]
        return carry

    lax.fori_loop(0, BLK // unroll, row, 0)


def _winv_body(d_ref, o_ref):
    o_ref[...] = (1.0 - ALPHA) / (d_ref[0] + d_ref[1] + 1.0)


def _winv(degp):
    _, n_acc, l = degp.shape
    return pl.pallas_call(
        _winv_body,
        out_shape=jax.ShapeDtypeStruct((n_acc, l), jnp.float32),
    )(degp)


def _make_agg1(n, dh, n_acc, k):
    rpt = n_acc // NS
    mesh = plsc.VectorSubcoreMesh(core_axis_name="c", subcore_axis_name="s")
    out_type = [
        jax.ShapeDtypeStruct((NC, n_acc, dh), jnp.float32),
        jax.ShapeDtypeStruct((NC, n_acc, L), jnp.float32),
    ]
    scratch = [
        pltpu.VMEM((NBUF, CHUNK), jnp.int32),
        pltpu.VMEM((k, CHUNK), jnp.int32),
        pltpu.VMEM((CHUNK, dh), jnp.float32),
        pltpu.VMEM((CHUNK, dh), jnp.float32),
        pltpu.VMEM((CHUNK, L), jnp.float32),
        pltpu.VMEM_SHARED((n_acc, dh), jnp.float32),
        pltpu.VMEM_SHARED((n, dh), jnp.float32),
        pltpu.VMEM_SHARED((n_acc, L), jnp.float32),
        pltpu.SemaphoreType.DMA,
        pltpu.SemaphoreType.DMA,
        pltpu.SemaphoreType.DMA,
    ]

    def body(h_hbm, src_hbm, dst_hbm, zerol_hbm, ones_hbm, p_hbm, deg_hbm,
             src_ring, dst_v, buf_a, buf_b, ones_v, acc_sh, table_sh, deg_sh,
             sem_a, sem_b, isem):
        c = lax.axis_index("c")
        s = lax.axis_index("s")
        @pl.when(s == 0)
        def _():
            pltpu.sync_copy(h_hbm.at[c, pl.ds(0, n)], table_sh)
        pltpu.sync_copy(dst_hbm.at[s], dst_v)
        row0 = s * rpt
        rows = pl.ds(row0, rpt)
        pltpu.sync_copy(h_hbm.at[c, rows], acc_sh.at[rows])
        pltpu.sync_copy(zerol_hbm.at[rows], deg_sh.at[rows])
        pltpu.sync_copy(ones_hbm, ones_v)

        _edge_loop(table_sh, acc_sh, src_hbm.at[s], src_ring, dst_v,
                   [buf_a, buf_b], [sem_a, sem_b], isem, k,
                   deg=(ones_v, deg_sh, c))

        pltpu.sync_copy(acc_sh.at[rows], p_hbm.at[c, rows])
        pltpu.sync_copy(deg_sh.at[rows], deg_hbm.at[c, rows])

    return pl.kernel(
        body,
        out_type=out_type,
        mesh=mesh,
        scratch_types=scratch,
        compiler_params=pltpu.CompilerParams(use_tc_tiling_on_sc=False),
    )


def _make_l2(n, dh, n_acc, k):
    rpt = n_acc // NS
    nblk = rpt // BLK
    mesh = plsc.VectorSubcoreMesh(core_axis_name="c", subcore_axis_name="s")
    out_type = [jax.ShapeDtypeStruct((n_acc, NC * dh), jnp.float32)]
    scratch = [
        pltpu.VMEM((NBUF, CHUNK), jnp.int32),
        pltpu.VMEM((k, CHUNK), jnp.int32),
        pltpu.VMEM((CHUNK, dh), jnp.float32),
        pltpu.VMEM((CHUNK, dh), jnp.float32),
        pltpu.VMEM((BLK, L), jnp.float32),
        pltpu.VMEM_SHARED((n_acc, dh), jnp.float32),
        pltpu.VMEM_SHARED((n_acc, dh), jnp.float32),
        pltpu.SemaphoreType.DMA,
        pltpu.SemaphoreType.DMA,
        pltpu.SemaphoreType.DMA,
    ]

    def body(p_hbm, w_hbm, h_hbm, src_hbm, dst_hbm, out_hbm,
             src_ring, dst_v, buf_a, buf_b, wbuf, acc_sh, table_sh,
             sem_a, sem_b, isem):
        c = lax.axis_index("c")
        s = lax.axis_index("s")
        pltpu.sync_copy(dst_hbm.at[s], dst_v)
        row0 = s * rpt
        for t in range(nblk):
            rows = pl.ds(row0 + t * BLK, BLK)
            pltpu.sync_copy(p_hbm.at[c, rows], buf_a)
            pltpu.sync_copy(h_hbm.at[c, rows], buf_b)
            pltpu.sync_copy(w_hbm.at[rows], wbuf)
            _blend(buf_a, buf_b, wbuf, dh)
            pltpu.sync_copy(buf_a, table_sh.at[rows])
            pltpu.sync_copy(buf_a, acc_sh.at[rows])

        _edge_loop(table_sh, acc_sh, src_hbm.at[s], src_ring, dst_v,
                   [buf_a, buf_b], [sem_a, sem_b], isem, k)

        for t in range(nblk):
            rows = pl.ds(row0 + t * BLK, BLK)
            pltpu.sync_copy(acc_sh.at[rows], buf_a)
            pltpu.sync_copy(h_hbm.at[c, rows], buf_b)
            pltpu.sync_copy(w_hbm.at[rows], wbuf)
            _blend(buf_a, buf_b, wbuf, dh)
            pltpu.sync_copy(buf_a, out_hbm.at[rows, pl.ds(c * dh, dh)])

    return pl.kernel(
        body,
        out_type=out_type,
        mesh=mesh,
        scratch_types=scratch,
        compiler_params=pltpu.CompilerParams(use_tc_tiling_on_sc=False),
    )


def kernel(x, edge_index, W0, b0, W1, b1):
    n, d = x.shape
    e = edge_index.shape[1]
    dh = d // 2

    n_acc = -(-(n + 1) // (NS * BLK)) * (NS * BLK)

    blk = 1000 if n % 1000 == 0 else 8
    h_split = _mlp(x, W0.T, b0.reshape(1, -1), W1.T, b1.reshape(1, -1), blk, n_acc)

    per_xfer = NS * CHUNK
    k = -(-e // per_xfer)
    k += k % NBUF
    e_pad = k * per_xfer
    pad = e_pad - e
    src = jnp.concatenate([edge_index[0], jnp.zeros((pad,), jnp.int32)])
    dst = jnp.concatenate([edge_index[1], jnp.full((pad,), n, jnp.int32)])
    src_p = src.reshape(NS, k, CHUNK)
    dst_p = dst.reshape(NS, k, CHUNK)

    zeros_l = jnp.zeros((n_acc, L), jnp.float32)
    ones_l = jnp.ones((CHUNK, L), jnp.float32)

    p1, degp = _make_agg1(n, dh, n_acc, k)(h_split, src_p, dst_p, zeros_l, ones_l)
    winv = _winv(degp)
    (x2s,) = _make_l2(n, dh, n_acc, k)(p1, winv, h_split, src_p, dst_p)
    return x2s[:n]
